# baseline (speedup 1.0000x reference)
import functools

import jax
import jax.numpy as jnp
from jax import lax
from jax.experimental import pallas as pl
from jax.experimental.pallas import tpu as pltpu
from jax.experimental.pallas import tpu_sc as plsc

F32 = jnp.float32

NC = 2
NS = 16
NW = NC * NS
LANES = 16

CHUNK = 100
MC = 80
ROWBLK = 64
I32 = jnp.int32


def _tca_body(x_ref, wz_ref, wh_ref, lzw_ref, lhw_ref, xw_ref):
    h = wz_ref.shape[0]
    az = lzw_ref[:, :h]
    ah = lhw_ref[:, :h]
    wz_eff = jnp.dot(az, wz_ref[...], preferred_element_type=F32)
    wh_eff = jnp.dot(ah, wh_ref[...], preferred_element_type=F32)
    wcat = jnp.concatenate([wz_eff, wh_eff], axis=0)
    xw_ref[...] = lax.dot_general(
        x_ref[...], wcat, (((1,), (1,)), ((), ())), preferred_element_type=F32)


def _tcc_body(sp_ref, di_ref, xw_ref, bz_ref, lzb_ref, bh_ref, lhb_ref,
              lzw_ref, lhw_ref, hn_ref):
    h = hn_ref.shape[1]
    dis = di_ref[:, :1]
    inv = di_ref[:, 1:2]
    s = sp_ref[0] + sp_ref[1]
    az = lzw_ref[:, :h]
    ah = lhw_ref[:, :h]
    bz2 = lax.dot_general(bz_ref[...], az, (((1,), (1,)), ((), ())),
                          preferred_element_type=F32) + lzb_ref[...]
    bh2 = lax.dot_general(bh_ref[...], ah, (((1,), (1,)), ((), ())),
                          preferred_element_type=F32) + lhb_ref[...]
    bcat = jnp.concatenate([bz2, bh2], axis=1)
    outcat = dis * s + xw_ref[...] * inv + bcat
    z = jax.nn.sigmoid(outcat[:, :h])
    ht = jnp.tanh(outcat[:, h:])
    hn_ref[...] = (1.0 - z) * ht


def _tcd_body(e_ref, w1_ref, b1_ref, w2_ref, b2_ref, o_ref):
    blk = e_ref.shape[0]
    h1 = lax.dot_general(e_ref[...], w1_ref[...], (((1,), (1,)), ((), ())),
                         preferred_element_type=F32) + b1_ref[...]
    h1 = jnp.maximum(h1, 0.0)
    lg = jnp.sum(h1 * w2_ref[...], axis=1, keepdims=True) + b2_ref[...]
    o_ref[...] = jax.nn.sigmoid(lg).reshape(1, 1, blk)


def _newton_rsqrt(x):
    i = plsc.bitcast(x, I32)
    i = jnp.int32(0x5F3759DF) - lax.shift_right_logical(i, 1)
    y = plsc.bitcast(i, F32)
    for _ in range(3):
        y = y * (1.5 - 0.5 * x * y * y)
    return y


def _scnode_body(npad, nchw, src4_hbm, dst4_hbm, xw_hbm, sp_hbm, di_hbm,
                 xws_hbm, srcb, dstb, rows0, rows1, xwbuf, sbuf, degv, tmp,
                 disbuf, invbuf, sem0, sem1, deg16, accum):
    c = lax.axis_index("c")
    s = lax.axis_index("s")
    slabn = npad // NS
    nvec = slabn // LANES
    nblk = slabn // ROWBLK
    nch2 = nchw // NC
    coff = c * npad

    pltpu.sync_copy(dst4_hbm.at[pl.ds(s * nchw, nchw)], dstb)
    pltpu.sync_copy(src4_hbm.at[pl.ds(s * nchw + c * nch2, nch2)], srcb)

    def obody(r, carry):
        for k in range(MC // LANES):
            sl = pl.ds(k * LANES, LANES)
            srcb[r, sl] = srcb[r, sl] + coff
        return carry

    lax.fori_loop(0, nch2, obody, 0)

    zeros = jnp.zeros((LANES,), F32)

    def zdeg(i, carry):
        degv[pl.ds(pl.multiple_of(i * LANES, LANES), LANES)] = zeros
        return carry

    lax.fori_loop(0, npad // LANES, zdeg, 0)
    ones = jnp.ones((LANES,), F32)

    def hrow(r, carry):
        for k in range(MC // LANES):
            idx = dstb[r, pl.ds(k * LANES, LANES)]
            plsc.addupdate_scatter(degv, [idx], ones)
        return carry

    lax.fori_loop(0, nchw, hrow, 0)
    pltpu.sync_copy(degv, deg16.at[s])
    plsc.subcore_barrier()

    def czero(v, carry):
        invbuf[pl.ds(pl.multiple_of(v * LANES, LANES), LANES)] = zeros
        return carry

    lax.fori_loop(0, nvec, czero, 0)
    for r in range(NS):
        pltpu.sync_copy(deg16.at[r, pl.ds(s * slabn, slabn)], tmp)

        def cacc(v, carry):
            sl = pl.ds(pl.multiple_of(v * LANES, LANES), LANES)
            invbuf[sl] = invbuf[sl] + tmp[sl]
            return carry

        lax.fori_loop(0, nvec, cacc, 0)

    def cbody(v, carry):
        sl = pl.ds(pl.multiple_of(v * LANES, LANES), LANES)
        acc = invbuf[sl] + 1.0
        y = _newton_rsqrt(acc)
        disbuf[sl] = y
        invbuf[sl] = y * y
        return carry

    lax.fori_loop(0, nvec, cbody, 0)

    @pl.when(c == 0)
    def _():
        pltpu.sync_copy(disbuf, di_hbm.at[0, pl.ds(s * slabn, slabn)])
        pltpu.sync_copy(invbuf, di_hbm.at[1, pl.ds(s * slabn, slabn)])

    def zrow(r, carry):
        for k in range(4):
            sbuf[r, pl.ds(k * LANES, LANES)] = zeros
        return carry

    lax.fori_loop(0, ROWBLK, zrow, 0)

    def bchunk(t, carry):
        gbase = s * slabn + t * ROWBLK
        pltpu.sync_copy(xw_hbm.at[pl.ds(gbase, ROWBLK)], xwbuf)

        def brow(r, carry2):
            ridx = jnp.full((LANES,), t * ROWBLK + r, I32)
            dsc = plsc.load_gather(disbuf, [ridx])
            for k in range(4):
                sl = pl.ds(k * LANES, LANES)
                xwbuf[r, sl] = xwbuf[r, sl] * dsc
            return carry2

        lax.fori_loop(0, ROWBLK, brow, 0)
        pltpu.sync_copy(xwbuf, xws_hbm.at[pl.ds(coff + gbase, ROWBLK)])
        pltpu.sync_copy(sbuf, accum.at[pl.ds(gbase, ROWBLK)])
        return carry

    lax.fori_loop(0, nblk, bchunk, 0)
    plsc.subcore_barrier()

    def fire(j, buf, sem):
        pltpu.async_copy(xws_hbm.at[srcb.at[j]], buf, sem)

    def mhalf(j, buf, sem):
        pltpu.make_async_copy(xws_hbm.at[srcb.at[j]], buf, sem).wait()
        pltpu.sync_copy(buf, accum.at[dstb.at[c * nch2 + j]], add=True)

    fire(0, rows0, sem0)
    fire(1, rows1, sem1)

    def mbody(i, carry):
        j = 2 * i
        mhalf(j, rows0, sem0)

        @pl.when(j + 2 < nch2)
        def _():
            fire(j + 2, rows0, sem0)

        mhalf(j + 1, rows1, sem1)

        @pl.when(j + 3 < nch2)
        def _():
            fire(j + 3, rows1, sem1)

        return carry

    lax.fori_loop(0, nch2 // 2, mbody, 0)
    if nch2 % 2 == 1:
        mhalf(nch2 - 1, rows0, sem0)
    plsc.subcore_barrier()

    def wchunk(t, carry):
        gbase = s * slabn + t * ROWBLK
        pltpu.sync_copy(accum.at[pl.ds(gbase, ROWBLK)], sbuf)
        pltpu.sync_copy(sbuf, sp_hbm.at[c, pl.ds(gbase, ROWBLK)])
        return carry

    lax.fori_loop(0, nblk, wchunk, 0)


def _scedge_body(ept, nch, h2, src2_hbm, dst2_hbm, hn_hbm, embs_hbm,
                 srcb, dstb, hb0, tb0, pb0, hb1, tb1, pb1,
                 semg0, semg1, semw0, semw1):
    c = lax.axis_index("c")
    s = lax.axis_index("s")
    wid = s * NC + c
    rowblk = pl.ds(wid * nch, nch)
    pltpu.sync_copy(src2_hbm.at[rowblk], srcb)
    pltpu.sync_copy(dst2_hbm.at[rowblk], dstb)

    def fireg(j, hb, tb, sem):
        pltpu.async_copy(hn_hbm.at[srcb.at[j]], hb, sem)
        pltpu.async_copy(hn_hbm.at[dstb.at[j]], tb, sem)

    def waitg(j, hb, tb, sem):
        pltpu.make_async_copy(hn_hbm.at[srcb.at[j]], hb, sem).wait()
        pltpu.make_async_copy(hn_hbm.at[dstb.at[j]], tb, sem).wait()

    fireg(0, hb0, tb0, semg0)
    fireg(1, hb1, tb1, semg1)

    def half(i, j, hb, tb, pb, semg, semw):
        base = wid * ept + j * CHUNK
        out_slc = embs_hbm.at[pl.ds(base, CHUNK)]
        waitg(j, hb, tb, semg)

        @pl.when(i > 0)
        def _():
            pltpu.make_async_copy(pb, out_slc, semw).wait()

        for r in range(CHUNK):
            for k in range(h2 // LANES):
                sl = pl.ds(k * LANES, LANES)
                pb[r, sl] = hb[r, sl] * tb[r, sl]
        pltpu.async_copy(pb, out_slc, semw)

    def body(i, carry):
        j = 2 * i
        half(i, j, hb0, tb0, pb0, semg0, semw0)

        @pl.when(i < nch // 2 - 1)
        def _():
            fireg(j + 2, hb0, tb0, semg0)

        half(i, j + 1, hb1, tb1, pb1, semg1, semw1)

        @pl.when(i < nch // 2 - 1)
        def _():
            fireg(j + 3, hb1, tb1, semg1)

        return carry

    lax.fori_loop(0, nch // 2, body, 0)
    tail = embs_hbm.at[pl.ds(wid * ept, CHUNK)]
    pltpu.make_async_copy(pb0, tail, semw0).wait()
    pltpu.make_async_copy(pb1, tail, semw1).wait()


def kernel(x, edge_index, Wz, bz, Wr, br, Wh, bh, lz_w, lz_b, lr_w, lr_b,
           lh_w, lh_b, mlp_w1, mlp_b1, mlp_w2, mlp_b2):
    n, f = x.shape
    h = Wz.shape[0]
    h2 = 2 * h
    e = edge_index.shape[1]
    assert e % (NW * CHUNK) == 0
    ept = e // NW
    nch = ept // CHUNK
    npad = ((n + NS * 8 - 1) // (NS * 8)) * (NS * 8)
    if npad != n:
        x = jnp.pad(x, ((0, npad - n), (0, 0)))

    src = edge_index[0]
    dst = edge_index[1]
    src2 = src.reshape(e // CHUNK, CHUNK)
    dst2 = dst.reshape(e // CHUNK, CHUNK)

    xw = pl.pallas_call(
        _tca_body,
        out_shape=jax.ShapeDtypeStruct((npad, h2), F32),
    )(x, Wz, Wh, lz_w, lh_w)

    mesh = plsc.VectorSubcoreMesh(core_axis_name="c", subcore_axis_name="s")
    src4 = edge_index[0].reshape(e // MC, MC)
    dst4 = edge_index[1].reshape(e // MC, MC)
    nchw = e // (NS * MC)
    sparts, di = pl.kernel(
        functools.partial(_scnode_body, npad, nchw),
        out_type=(jax.ShapeDtypeStruct((NC, npad, h2), F32),
                  jax.ShapeDtypeStruct((2, npad), F32)),
        mesh=mesh,
        compiler_params=pltpu.CompilerParams(needs_layout_passes=False, use_tc_tiling_on_sc=False),
        scratch_types=[
            pltpu.HBM((4 * NC * npad, h2), F32),
            pltpu.VMEM((nchw // NC, MC), I32),
            pltpu.VMEM((nchw, MC), I32),
            pltpu.VMEM((MC, h2), F32),
            pltpu.VMEM((MC, h2), F32),
            pltpu.VMEM((ROWBLK, h2), F32),
            pltpu.VMEM((ROWBLK, h2), F32),
            pltpu.VMEM((npad,), F32),
            pltpu.VMEM((npad // NS,), F32),
            pltpu.VMEM((npad // NS,), F32),
            pltpu.VMEM((npad // NS,), F32),
            pltpu.SemaphoreType.DMA,
            pltpu.SemaphoreType.DMA,
            pltpu.VMEM_SHARED((NS, npad), F32),
            pltpu.VMEM_SHARED((npad, h2), F32),
        ],
    )(src4, dst4, xw)
    din = di.T

    hn = pl.pallas_call(
        _tcc_body,
        out_shape=jax.ShapeDtypeStruct((npad, h), F32),
    )(sparts, din, xw, bz.reshape(1, h), lz_b.reshape(1, h),
      bh.reshape(1, h), lh_b.reshape(1, h), lz_w, lh_w)

    blk = 16000

    def edge_half(srch, dsth, eh):
        nchh = (eh // NC // NS) // CHUNK
        return pl.kernel(
            functools.partial(_scedge_body, eh // (NC * NS), nchh, h),
            out_type=jax.ShapeDtypeStruct((eh, h), F32),
            mesh=mesh,
            compiler_params=pltpu.CompilerParams(needs_layout_passes=False, use_tc_tiling_on_sc=False),
            scratch_types=[
                pltpu.VMEM((nchh, CHUNK), jnp.int32),
                pltpu.VMEM((nchh, CHUNK), jnp.int32),
                pltpu.VMEM((CHUNK, h), F32),
                pltpu.VMEM((CHUNK, h), F32),
                pltpu.VMEM((CHUNK, h), F32),
                pltpu.VMEM((CHUNK, h), F32),
                pltpu.VMEM((CHUNK, h), F32),
                pltpu.VMEM((CHUNK, h), F32),
                pltpu.SemaphoreType.DMA,
                pltpu.SemaphoreType.DMA,
                pltpu.SemaphoreType.DMA,
                pltpu.SemaphoreType.DMA,
            ],
        )(srch, dsth, hn)

    def mlp_half(embsh, eh):
        return pl.pallas_call(
            _tcd_body,
            grid=(eh // blk,),
            in_specs=[
                pl.BlockSpec((blk, h), lambda i: (i, 0)),
                pl.BlockSpec((h, h), lambda i: (0, 0)),
                pl.BlockSpec((1, h), lambda i: (0, 0)),
                pl.BlockSpec((1, h), lambda i: (0, 0)),
                pl.BlockSpec((1, 1), lambda i: (0, 0)),
            ],
            out_specs=pl.BlockSpec((1, 1, blk), lambda i: (i, 0, 0)),
            out_shape=jax.ShapeDtypeStruct((eh // blk, 1, blk), F32),
        )(embsh, mlp_w1, mlp_b1.reshape(1, h), mlp_w2, mlp_b2.reshape(1, 1))

    eh = e // 2
    nrow = e // CHUNK
    embs1 = edge_half(src2[:nrow // 2], dst2[:nrow // 2], eh)
    out1 = mlp_half(embs1, eh)
    embs2 = edge_half(src2[nrow // 2:], dst2[nrow // 2:], eh)
    out2 = mlp_half(embs2, eh)
    return jnp.concatenate([out1.reshape(eh, 1), out2.reshape(eh, 1)], axis=0)


# speedup vs baseline: 1.0114x; 1.0114x over previous
import functools

import jax
import jax.numpy as jnp
from jax import lax
from jax.experimental import pallas as pl
from jax.experimental.pallas import tpu as pltpu
from jax.experimental.pallas import tpu_sc as plsc

F32 = jnp.float32

NC = 2
NS = 16
NW = NC * NS
LANES = 16

CHUNK = 100


def _tca_body(x_ref, wz_ref, wh_ref, lzw_ref, lhw_ref, xw_ref):
    h = wz_ref.shape[0]
    az = lzw_ref[:, :h]
    ah = lhw_ref[:, :h]
    wz_eff = jnp.dot(az, wz_ref[...], preferred_element_type=F32)
    wh_eff = jnp.dot(ah, wh_ref[...], preferred_element_type=F32)
    wcat = jnp.concatenate([wz_eff, wh_eff], axis=0)
    xw_ref[...] = lax.dot_general(
        x_ref[...], wcat, (((1,), (1,)), ((), ())), preferred_element_type=F32)


def _tcb_body(degpt_ref, xw_ref, xws_ref):
    deg = jnp.sum(degpt_ref[...], axis=1, keepdims=True) + 1.0
    dis = lax.rsqrt(deg)
    xws_ref[...] = xw_ref[...] * dis


def _tcc_body(sp_ref, degpt_ref, xw_ref, bz_ref, lzb_ref, bh_ref, lhb_ref,
              lzw_ref, lhw_ref, hn_ref):
    h = hn_ref.shape[1]
    deg = jnp.sum(degpt_ref[...], axis=1, keepdims=True) + 1.0
    dis = lax.rsqrt(deg)
    s = sp_ref[0] + sp_ref[1]
    az = lzw_ref[:, :h]
    ah = lhw_ref[:, :h]
    bz2 = lax.dot_general(bz_ref[...], az, (((1,), (1,)), ((), ())),
                          preferred_element_type=F32) + lzb_ref[...]
    bh2 = lax.dot_general(bh_ref[...], ah, (((1,), (1,)), ((), ())),
                          preferred_element_type=F32) + lhb_ref[...]
    bcat = jnp.concatenate([bz2, bh2], axis=1)
    outcat = dis * s + xw_ref[...] / deg + bcat
    z = jax.nn.sigmoid(outcat[:, :h])
    ht = jnp.tanh(outcat[:, h:])
    hn_ref[...] = (1.0 - z) * ht


def _tcd_body(e_ref, w1_ref, b1_ref, w2_ref, b2_ref, o_ref):
    blk = e_ref.shape[0]
    h1 = lax.dot_general(e_ref[...], w1_ref[...], (((1,), (1,)), ((), ())),
                         preferred_element_type=F32) + b1_ref[...]
    h1 = jnp.maximum(h1, 0.0)
    lg = jnp.sum(h1 * w2_ref[...], axis=1, keepdims=True) + b2_ref[...]
    o_ref[...] = jax.nn.sigmoid(lg).reshape(1, 1, blk)


def _make_mesh():
    return plsc.VectorSubcoreMesh(core_axis_name="c", subcore_axis_name="s")


def _scdeg_body(npad, ept, dst_hbm, degp_hbm, dstv, degv):
    c = lax.axis_index("c")
    s = lax.axis_index("s")
    wid = s * NC + c
    zeros = jnp.zeros((LANES,), F32)

    def zbody(i, carry):
        degv[pl.ds(pl.multiple_of(i * LANES, LANES), LANES)] = zeros
        return carry

    lax.fori_loop(0, npad // LANES, zbody, 0)

    base = pl.multiple_of(wid * ept, 8)
    pltpu.sync_copy(dst_hbm.at[pl.ds(base, ept)], dstv)
    ones = jnp.ones((LANES,), F32)

    def body(i, carry):
        idx = dstv[pl.ds(pl.multiple_of(i * LANES, LANES), LANES)]
        plsc.addupdate_scatter(degv, [idx], ones)
        return carry

    lax.fori_loop(0, ept // LANES, body, 0)
    pltpu.sync_copy(degv, degp_hbm.at[pl.ds(pl.multiple_of(wid * npad, 8), npad)])


def _scmsg_body(npad, nch, src2_hbm, dst2_hbm, xws_hbm, z64_hbm, sp_hbm,
                srcb, dstb, rows0, rows1, vbuf, accum, sem0, sem1):
    c = lax.axis_index("c")
    s = lax.axis_index("s")
    wid = s * NC + c
    slab = pl.ds(s * (npad // NS), npad // NS)
    pltpu.sync_copy(z64_hbm.at[slab], vbuf)
    pltpu.sync_copy(vbuf, accum.at[slab])
    rowblk = pl.ds(wid * nch, nch)
    pltpu.sync_copy(src2_hbm.at[rowblk], srcb)
    pltpu.sync_copy(dst2_hbm.at[rowblk], dstb)
    plsc.subcore_barrier()

    def fire(j, buf, sem):
        pltpu.async_copy(xws_hbm.at[srcb.at[j]], buf, sem)

    fire(0, rows0, sem0)
    fire(1, rows1, sem1)

    def body(i, carry):
        j = 2 * i
        pltpu.make_async_copy(xws_hbm.at[srcb.at[j]], rows0, sem0).wait()
        pltpu.sync_copy(rows0, accum.at[dstb.at[j]], add=True)

        @pl.when(i < nch // 2 - 1)
        def _():
            fire(j + 2, rows0, sem0)

        pltpu.make_async_copy(xws_hbm.at[srcb.at[j + 1]], rows1, sem1).wait()
        pltpu.sync_copy(rows1, accum.at[dstb.at[j + 1]], add=True)

        @pl.when(i < nch // 2 - 1)
        def _():
            fire(j + 3, rows1, sem1)

        return carry

    lax.fori_loop(0, nch // 2, body, 0)
    plsc.subcore_barrier()
    pltpu.sync_copy(accum.at[slab], vbuf)
    pltpu.sync_copy(vbuf, sp_hbm.at[c, slab])


def _scedge_body(ept, nch, h2, src2_hbm, dst2_hbm, hn_hbm, embs_hbm,
                 srcb, dstb, hb0, tb0, pb0, hb1, tb1, pb1,
                 semg0, semg1, semw0, semw1):
    c = lax.axis_index("c")
    s = lax.axis_index("s")
    wid = s * NC + c
    rowblk = pl.ds(wid * nch, nch)
    pltpu.sync_copy(src2_hbm.at[rowblk], srcb)
    pltpu.sync_copy(dst2_hbm.at[rowblk], dstb)

    def fireg(j, hb, tb, sem):
        pltpu.async_copy(hn_hbm.at[srcb.at[j]], hb, sem)
        pltpu.async_copy(hn_hbm.at[dstb.at[j]], tb, sem)

    def waitg(j, hb, tb, sem):
        pltpu.make_async_copy(hn_hbm.at[srcb.at[j]], hb, sem).wait()
        pltpu.make_async_copy(hn_hbm.at[dstb.at[j]], tb, sem).wait()

    fireg(0, hb0, tb0, semg0)
    fireg(1, hb1, tb1, semg1)

    def half(i, j, hb, tb, pb, semg, semw):
        base = wid * ept + j * CHUNK
        out_slc = embs_hbm.at[pl.ds(base, CHUNK)]
        waitg(j, hb, tb, semg)

        @pl.when(i > 0)
        def _():
            pltpu.make_async_copy(pb, out_slc, semw).wait()

        for r in range(CHUNK):
            for k in range(h2 // LANES):
                sl = pl.ds(k * LANES, LANES)
                pb[r, sl] = hb[r, sl] * tb[r, sl]
        pltpu.async_copy(pb, out_slc, semw)

    def body(i, carry):
        j = 2 * i
        half(i, j, hb0, tb0, pb0, semg0, semw0)

        @pl.when(i < nch // 2 - 1)
        def _():
            fireg(j + 2, hb0, tb0, semg0)

        half(i, j + 1, hb1, tb1, pb1, semg1, semw1)

        @pl.when(i < nch // 2 - 1)
        def _():
            fireg(j + 3, hb1, tb1, semg1)

        return carry

    lax.fori_loop(0, nch // 2, body, 0)
    tail = embs_hbm.at[pl.ds(wid * ept, CHUNK)]
    pltpu.make_async_copy(pb0, tail, semw0).wait()
    pltpu.make_async_copy(pb1, tail, semw1).wait()


def kernel(x, edge_index, Wz, bz, Wr, br, Wh, bh, lz_w, lz_b, lr_w, lr_b,
           lh_w, lh_b, mlp_w1, mlp_b1, mlp_w2, mlp_b2):
    n, f = x.shape
    h = Wz.shape[0]
    h2 = 2 * h
    e = edge_index.shape[1]
    assert e % (NW * CHUNK) == 0
    ept = e // NW
    nch = ept // CHUNK
    npad = ((n + NS * 8 - 1) // (NS * 8)) * (NS * 8)
    if npad != n:
        x = jnp.pad(x, ((0, npad - n), (0, 0)))

    src = edge_index[0]
    dst = edge_index[1]
    src2 = src.reshape(e // CHUNK, CHUNK)
    dst2 = dst.reshape(e // CHUNK, CHUNK)

    xw = pl.pallas_call(
        _tca_body,
        out_shape=jax.ShapeDtypeStruct((npad, h2), F32),
    )(x, Wz, Wh, lz_w, lh_w)

    mesh = _make_mesh()
    degp = pl.kernel(
        functools.partial(_scdeg_body, npad, ept),
        out_type=jax.ShapeDtypeStruct((NW * npad,), F32),
        mesh=mesh,
        compiler_params=pltpu.CompilerParams(needs_layout_passes=False, use_tc_tiling_on_sc=False),
        scratch_types=[
            pltpu.VMEM((ept,), jnp.int32),
            pltpu.VMEM((npad,), F32),
        ],
    )(dst)
    degpt = degp.reshape(NW, npad).T

    xws = pl.pallas_call(
        _tcb_body,
        out_shape=jax.ShapeDtypeStruct((npad, h2), F32),
    )(degpt, xw)

    z64 = jnp.zeros((npad, h2), F32)
    sparts = pl.kernel(
        functools.partial(_scmsg_body, npad, nch),
        out_type=jax.ShapeDtypeStruct((NC, npad, h2), F32),
        mesh=mesh,
        compiler_params=pltpu.CompilerParams(needs_layout_passes=False, use_tc_tiling_on_sc=False),
        scratch_types=[
            pltpu.VMEM((nch, CHUNK), jnp.int32),
            pltpu.VMEM((nch, CHUNK), jnp.int32),
            pltpu.VMEM((CHUNK, h2), F32),
            pltpu.VMEM((CHUNK, h2), F32),
            pltpu.VMEM((npad // NS, h2), F32),
            pltpu.VMEM_SHARED((npad, h2), F32),
            pltpu.SemaphoreType.DMA,
            pltpu.SemaphoreType.DMA,
        ],
    )(src2, dst2, xws, z64)

    hn = pl.pallas_call(
        _tcc_body,
        out_shape=jax.ShapeDtypeStruct((npad, h), F32),
    )(sparts, degpt, xw, bz.reshape(1, h), lz_b.reshape(1, h),
      bh.reshape(1, h), lh_b.reshape(1, h), lz_w, lh_w)

    blk = 16000

    def edge_half(srch, dsth, eh):
        nchh = (eh // NC // NS) // CHUNK
        return pl.kernel(
            functools.partial(_scedge_body, eh // (NC * NS), nchh, h),
            out_type=jax.ShapeDtypeStruct((eh, h), F32),
            mesh=mesh,
            compiler_params=pltpu.CompilerParams(needs_layout_passes=False, use_tc_tiling_on_sc=False),
            scratch_types=[
                pltpu.VMEM((nchh, CHUNK), jnp.int32),
                pltpu.VMEM((nchh, CHUNK), jnp.int32),
                pltpu.VMEM((CHUNK, h), F32),
                pltpu.VMEM((CHUNK, h), F32),
                pltpu.VMEM((CHUNK, h), F32),
                pltpu.VMEM((CHUNK, h), F32),
                pltpu.VMEM((CHUNK, h), F32),
                pltpu.VMEM((CHUNK, h), F32),
                pltpu.SemaphoreType.DMA,
                pltpu.SemaphoreType.DMA,
                pltpu.SemaphoreType.DMA,
                pltpu.SemaphoreType.DMA,
            ],
        )(srch, dsth, hn)

    def mlp_half(embsh, eh):
        return pl.pallas_call(
            _tcd_body,
            grid=(eh // blk,),
            in_specs=[
                pl.BlockSpec((blk, h), lambda i: (i, 0)),
                pl.BlockSpec((h, h), lambda i: (0, 0)),
                pl.BlockSpec((1, h), lambda i: (0, 0)),
                pl.BlockSpec((1, h), lambda i: (0, 0)),
                pl.BlockSpec((1, 1), lambda i: (0, 0)),
            ],
            out_specs=pl.BlockSpec((1, 1, blk), lambda i: (i, 0, 0)),
            out_shape=jax.ShapeDtypeStruct((eh // blk, 1, blk), F32),
        )(embsh, mlp_w1, mlp_b1.reshape(1, h), mlp_w2, mlp_b2.reshape(1, 1))

    eh = e // 2
    nrow = e // CHUNK
    embs1 = edge_half(src2[:nrow // 2], dst2[:nrow // 2], eh)
    out1 = mlp_half(embs1, eh)
    embs2 = edge_half(src2[nrow // 2:], dst2[nrow // 2:], eh)
    out2 = mlp_half(embs2, eh)
    return jnp.concatenate([out1.reshape(eh, 1), out2.reshape(eh, 1)], axis=0)
